# Initial kernel scaffold; baseline (speedup 1.0000x reference)
import jax
import jax.numpy as jnp
from jax.experimental import pallas as pl


def kernel(x_num, cat_idx, mean, std):
    raise NotImplementedError("write your pallas kernel here")


# speedup vs baseline: 1.4871x; 1.4871x over previous
import functools

import jax
import jax.numpy as jnp
from jax import lax
from jax.experimental import pallas as pl
from jax.experimental.pallas import tpu as pltpu
from jax.experimental.pallas import tpu_sc as plsc

NUM_TOKENS = 1000
N_FIELDS = 26
N_NUMERIC = 13
BATCH = 1024
WIDTH = N_NUMERIC + N_FIELDS * NUM_TOKENS
WIDTH_PAD = ((WIDTH + 15) // 16) * 16
L = 16
NW = 32
ROWS_PER_W = BATCH // NW
NBUF = 4


def _sc_body(x_hbm, cat_hbm, mean_hbm, std_hbm, out_hbm,
             x_v, cat_v, ms_v, bufs_and_sems):
    bufs = bufs_and_sems[:NBUF]
    sems = bufs_and_sems[NBUF:]
    cid = lax.axis_index("c")
    sid = lax.axis_index("s")
    wid = sid * 2 + cid
    base = wid * ROWS_PER_W

    pltpu.sync_copy(x_hbm.at[pl.ds(base, ROWS_PER_W)], x_v)
    pltpu.sync_copy(cat_hbm.at[pl.ds(base, ROWS_PER_W)], cat_v)
    pltpu.sync_copy(mean_hbm, ms_v.at[0])
    pltpu.sync_copy(std_hbm, ms_v.at[1])

    zeros = jnp.zeros((L,), jnp.float32)
    ones = jnp.ones((L,), jnp.float32)
    iota = lax.iota(jnp.int32, L)
    offs0 = N_NUMERIC + iota * NUM_TOKENS
    offs1 = N_NUMERIC + (iota + L) * NUM_TOKENS
    mask1 = iota < (N_FIELDS - L)

    def _zero(i, _):
        for b in range(NBUF):
            bufs[b][pl.ds(i * L, L)] = zeros
        return 0
    lax.fori_loop(0, WIDTH_PAD // L, _zero, 0)

    meanv = ms_v[0, :]
    stdv = ms_v[1, :]

    def cols_of(r):
        c0 = cat_v[r, pl.ds(0, L)] + offs0
        c1 = cat_v[r, pl.ds(L, L)] + offs1
        return c0, c1

    def build(r, b):
        buf = bufs[b]
        buf[pl.ds(0, L)] = (x_v[r, :] - meanv) / stdv
        c0, c1 = cols_of(r)
        plsc.store_scatter(buf, [c0], ones)
        plsc.store_scatter(buf, [c1], ones, mask=mask1)
        pltpu.async_copy(buf.at[pl.ds(0, WIDTH)], out_hbm.at[base + r], sems[b])

    def clear(r, b):
        buf = bufs[b]
        c0, c1 = cols_of(r)
        plsc.store_scatter(buf, [c0], zeros)
        plsc.store_scatter(buf, [c1], zeros, mask=mask1)
        buf[pl.ds(0, L)] = zeros

    for r in range(ROWS_PER_W):
        b = r % NBUF
        if r >= NBUF:
            pltpu.make_async_copy(bufs[b].at[pl.ds(0, WIDTH)],
                                  out_hbm.at[base + r - NBUF], sems[b]).wait()
            clear(r - NBUF, b)
        build(r, b)
    for r in range(ROWS_PER_W - NBUF, ROWS_PER_W):
        b = r % NBUF
        pltpu.make_async_copy(bufs[b].at[pl.ds(0, WIDTH)],
                              out_hbm.at[base + r], sems[b]).wait()


@jax.jit
def kernel(x_num, cat_idx, mean, std):
    x_pad = jnp.zeros((BATCH, L), jnp.float32).at[:, :N_NUMERIC].set(x_num)
    cat_pad = jnp.zeros((BATCH, 2 * L), jnp.int32).at[:, :N_FIELDS].set(
        cat_idx.astype(jnp.int32))
    mean_pad = jnp.zeros((L,), jnp.float32).at[:N_NUMERIC].set(mean)
    std_pad = jnp.ones((L,), jnp.float32).at[:N_NUMERIC].set(std)

    mesh = plsc.VectorSubcoreMesh(core_axis_name="c", subcore_axis_name="s")
    f = pl.kernel(
        _sc_body,
        out_type=jax.ShapeDtypeStruct((BATCH, WIDTH), jnp.float32),
        mesh=mesh,
        compiler_params=pltpu.CompilerParams(needs_layout_passes=False,
                                             use_tc_tiling_on_sc=False),
        scratch_types=[
            pltpu.VMEM((ROWS_PER_W, L), jnp.float32),
            pltpu.VMEM((ROWS_PER_W, 2 * L), jnp.int32),
            pltpu.VMEM((2, L), jnp.float32),
            [pltpu.VMEM((WIDTH_PAD,), jnp.float32) for _ in range(NBUF)]
            + [pltpu.SemaphoreType.DMA for _ in range(NBUF)],
        ],
    )
    return f(x_pad, cat_pad, mean_pad, std_pad)


# speedup vs baseline: 2.1746x; 1.4623x over previous
import jax
import jax.numpy as jnp
from jax import lax
from jax.experimental import pallas as pl
from jax.experimental.pallas import tpu as pltpu
from jax.experimental.pallas import tpu_sc as plsc

NUM_TOKENS = 1000
N_FIELDS = 26
N_NUMERIC = 13
BATCH = 1024
WIDTH = N_NUMERIC + N_FIELDS * NUM_TOKENS
L = 16
GROUPS_PER_W = 4
CHUNK_W = 51 * 128
NCHUNK = 4
Q3_MAIN_W = 50 * 128
TAIL_OFF = 3 * CHUNK_W + Q3_MAIN_W
TAIL_W = WIDTH - TAIL_OFF
NB = 2
USE_C0 = (True, True, True, False)
USE_C1 = (False, False, True, True)


def _sc_body(x_hbm, cat_hbm, mean_hbm, std_hbm, out_hbm,
             x_v, cat_v, m_v, s_v, stash, tail, bufs_and_sems):
    bufs = bufs_and_sems[:NB]
    sems = bufs_and_sems[NB:NB + NB]
    tsem = bufs_and_sems[2 * NB]
    wid = lax.axis_index("s") * 2 + lax.axis_index("c")
    rbase = wid * 32

    pltpu.sync_copy(x_hbm.at[pl.ds(rbase * 16, 32 * 16)], x_v)
    pltpu.sync_copy(cat_hbm.at[pl.ds(rbase * 32, 32 * 32)], cat_v)
    pltpu.sync_copy(mean_hbm, m_v)
    pltpu.sync_copy(std_hbm, s_v)

    zeros = jnp.zeros((L,), jnp.float32)
    ones = jnp.ones((L,), jnp.float32)
    iota = lax.iota(jnp.int32, L)
    offs0 = N_NUMERIC + iota * NUM_TOKENS
    offs1 = N_NUMERIC + (iota + L) * NUM_TOKENS
    valid1 = iota < (N_FIELDS - L)

    def _zero(j, _):
        for b in range(NB):
            for r in range(8):
                bufs[b][r, pl.ds(j * L, L)] = zeros
        return 0
    lax.fori_loop(0, CHUNK_W // L, _zero, 0)
    for r in range(8):
        tail[r, pl.ds(0, L)] = zeros
        tail[r, pl.ds(TAIL_W - L, L)] = zeros

    def emit(g, q, b, build):
        buf = bufs[b]
        lo = q * CHUNK_W
        hi = lo + (CHUNK_W if q < 3 else Q3_MAIN_W)
        for r in range(8):
            R = g * 8 + r
            rr = jnp.full((L,), r, jnp.int32)
            if q == 0:
                if build:
                    val = (x_v[pl.ds(R * 16, L)] - m_v[...]) / s_v[...]
                else:
                    val = zeros
                buf[r, pl.ds(0, L)] = val
            if build:
                val = ones
                if USE_C0[q]:
                    c0 = cat_v[pl.ds(R * 32, L)] + offs0
                    in0 = (c0 >= lo) & (c0 < hi)
                    k0 = jnp.where(in0, c0 - lo, 0)
                    stash[b, 2 * r, :] = k0
                    plsc.store_scatter(buf, [rr, k0], val, mask=in0)
                if USE_C1[q]:
                    c1 = cat_v[pl.ds(R * 32 + L, L)] + offs1
                    in1 = (c1 >= lo) & (c1 < hi) & valid1
                    k1 = jnp.where(in1, c1 - lo, 0)
                    stash[b, 2 * r + 1, :] = k1
                    plsc.store_scatter(buf, [rr, k1], val, mask=in1)
                if q == 3:
                    it = (c1 >= TAIL_OFF) & valid1
                    kt = jnp.where(it, c1 - TAIL_OFF, 0)
                    stash[2, r, :] = kt
                    plsc.store_scatter(tail, [rr, kt], val, mask=it)
            else:
                if USE_C0[q]:
                    plsc.store_scatter(buf, [rr, stash[b, 2 * r, :]], zeros)
                if USE_C1[q]:
                    plsc.store_scatter(buf, [rr, stash[b, 2 * r + 1, :]], zeros)
                if q == 3:
                    plsc.store_scatter(tail, [rr, stash[2, r, :]], zeros)

    def dma(t):
        g, q, b = t >> 2, t & 3, t & 1
        row0 = rbase + g * 8
        if q < 3:
            return [pltpu.make_async_copy(
                bufs[b],
                out_hbm.at[pl.ds(row0, 8), pl.ds(q * CHUNK_W, CHUNK_W)],
                sems[b])]
        return [
            pltpu.make_async_copy(
                bufs[b].at[:, pl.ds(0, Q3_MAIN_W)],
                out_hbm.at[pl.ds(row0, 8), pl.ds(q * CHUNK_W, Q3_MAIN_W)],
                sems[b]),
            pltpu.make_async_copy(
                tail, out_hbm.at[pl.ds(row0, 8), pl.ds(TAIL_OFF, TAIL_W)],
                tsem),
        ]

    NT = GROUPS_PER_W * NCHUNK
    for t in range(NT):
        g, q, b = t >> 2, t & 3, t & 1
        if t >= NB:
            for c in dma(t - NB):
                c.wait()
            emit((t - NB) >> 2, (t - NB) & 3, b, False)
        emit(g, q, b, True)
        for c in dma(t):
            c.start()
    for t in range(NT - NB, NT):
        for c in dma(t):
            c.wait()


@jax.jit
def kernel(x_num, cat_idx, mean, std):
    x_flat = jnp.pad(x_num, ((0, 0), (0, 3))).reshape(-1)
    cat_flat = jnp.pad(cat_idx.astype(jnp.int32), ((0, 0), (0, 6))).reshape(-1)
    mean16 = jnp.pad(mean, (0, 3))
    std16 = jnp.pad(std, (0, 3), constant_values=1.0)

    mesh = plsc.VectorSubcoreMesh(core_axis_name="c", subcore_axis_name="s")
    f = pl.kernel(
        _sc_body,
        out_type=jax.ShapeDtypeStruct((BATCH, WIDTH), jnp.float32),
        mesh=mesh,
        compiler_params=pltpu.CompilerParams(needs_layout_passes=False,
                                             use_tc_tiling_on_sc=True),
        scratch_types=[
            pltpu.VMEM((32 * 16,), jnp.float32),
            pltpu.VMEM((32 * 32,), jnp.int32),
            pltpu.VMEM((L,), jnp.float32),
            pltpu.VMEM((L,), jnp.float32),
            pltpu.VMEM((3, 16, L), jnp.int32),
            pltpu.VMEM((8, TAIL_W), jnp.float32),
            [pltpu.VMEM((8, CHUNK_W), jnp.float32) for _ in range(NB)]
            + [pltpu.SemaphoreType.DMA for _ in range(NB)]
            + [pltpu.SemaphoreType.DMA],
        ],
    )
    return f(x_flat, cat_flat, mean16, std16)


# speedup vs baseline: 3.1794x; 1.4621x over previous
import jax
import jax.numpy as jnp
from jax import lax
from jax.experimental import pallas as pl
from jax.experimental.pallas import tpu as pltpu
from jax.experimental.pallas import tpu_sc as plsc

NUM_TOKENS = 1000
N_FIELDS = 26
N_NUMERIC = 13
BATCH = 1024
WIDTH = N_NUMERIC + N_FIELDS * NUM_TOKENS
L = 16
NW = 32
SLAB = 816
CH = 40
NFULL = 20
REM = SLAB - NFULL * CH
NJ = BATCH // L
W31_FULL = 17
P32_LO = (NW - 1) * SLAB + W31_FULL * CH
P5_LO = P32_LO + 32
NB = 2


def _sc_body(xt_hbm, catt_hbm, mean_hbm, std_hbm, out_hbm,
             xrow_v, cat_v, m_s, s_s, stash, tail5, bufs_and_sems):
    bufs = bufs_and_sems[:NB]
    sems = bufs_and_sems[NB:NB + NB]
    tsem = bufs_and_sems[2 * NB]
    wid = lax.axis_index("s") * 2 + lax.axis_index("c")
    slab_lo = wid * SLAB

    f0 = jnp.maximum((slab_lo - N_NUMERIC) // NUM_TOKENS, 0)
    pltpu.sync_copy(catt_hbm.at[pl.ds(f0 * BATCH, 2 * BATCH)], cat_v)
    pltpu.sync_copy(mean_hbm, m_s)
    pltpu.sync_copy(std_hbm, s_s)

    zeros = jnp.zeros((L,), jnp.float32)
    ones = jnp.ones((L,), jnp.float32)
    iota = lax.iota(jnp.int32, L)
    cb0 = N_NUMERIC + f0 * NUM_TOKENS
    cb1 = cb0 + NUM_TOKENS

    def _zero(i, _):
        r = i >> 6
        j = (i & 63) * L
        for b in range(NB):
            bufs[b][r, pl.ds(j, L)] = zeros
        return 0
    lax.fori_loop(0, CH * NJ, _zero, 0)

    def _zero5(i, _):
        tail5[i >> 6, pl.ds((i & 63) * L, L)] = zeros
        return 0
    lax.fori_loop(0, 5 * NJ, _zero5, 0)

    def numeric(b, build):
        @pl.when(wid == 0)
        def _():
            def _row(c, _):
                if build:
                    pltpu.sync_copy(xt_hbm.at[pl.ds(c * BATCH, BATCH)],
                                    xrow_v)
                mc = m_s[pl.ds(c * L, L)]
                sc = s_s[pl.ds(c * L, L)]

                def _col(j, _):
                    if build:
                        v = (xrow_v[pl.ds(j * L, L)] - mc) / sc
                    else:
                        v = zeros
                    bufs[b][c, pl.ds(j * L, L)] = v
                    return 0
                lax.fori_loop(0, NJ, _col, 0)
                return 0
            lax.fori_loop(0, N_NUMERIC, _row, 0)

    def scatter_scan(buf, lo, hi, stash_b):
        def _s(j, _):
            l = j >> 6
            jj = j & 63
            cbase = jnp.where(l == 0, cb0, cb1)
            c = cbase + cat_v[pl.ds(l * BATCH + jj * L, L)]
            m = (c >= lo) & (c < hi)
            krow = jnp.where(m, c - lo, 0)
            if stash_b is not None:
                stash[stash_b, j, :] = krow
            plsc.store_scatter(buf, [krow, jj * L + iota], ones, mask=m)
            return 0
        lax.fori_loop(0, 2 * NJ, _s, 0)

    def clear(b):
        def _c(j, _):
            plsc.store_scatter(
                bufs[b], [stash[b, j, :], (j & 63) * L + iota], zeros)
            return 0
        lax.fori_loop(0, 2 * NJ, _c, 0)

    def fire(b, lo):
        return pltpu.make_async_copy(
            bufs[b], out_hbm.at[pl.ds(lo, CH), :], sems[b])

    def step(k, b):
        lo = slab_lo + k * CH
        fire(b, lo - NB * CH).wait()
        clear(b)
        scatter_scan(bufs[b], lo, lo + CH, b)
        fire(b, lo).start()

    numeric(0, build=True)
    scatter_scan(bufs[0], slab_lo, slab_lo + CH, 0)
    fire(0, slab_lo).start()
    scatter_scan(bufs[1], slab_lo + CH, slab_lo + 2 * CH, 1)
    fire(1, slab_lo + CH).start()

    def _pair(k2, _):
        for s in range(NB):
            k = 2 * k2 + s
            lo = slab_lo + k * CH
            fire(s, lo - NB * CH).wait()
            clear(s)
            if s == 0:
                @pl.when(k2 == 1)
                def _():
                    numeric(0, build=False)
            scatter_scan(bufs[s], lo, lo + CH, s)
            fire(s, lo).start()
        return 0
    lax.fori_loop(1, 8, _pair, 0)

    @pl.when(wid < NW - 1)
    def _():
        for k in range(16, NFULL):
            step(k, k & 1)
        lo = slab_lo + NFULL * CH
        fire(0, lo - NB * CH).wait()
        clear(0)
        scatter_scan(bufs[0], lo, lo + REM, 0)
        rem = pltpu.make_async_copy(
            bufs[0].at[pl.ds(0, REM), :],
            out_hbm.at[pl.ds(lo, REM), :], sems[0])
        rem.start()
        fire(1, slab_lo + (NFULL - 1) * CH).wait()
        rem.wait()

    @pl.when(wid == NW - 1)
    def _():
        step(16, 0)
        fire(1, slab_lo + 15 * CH).wait()
        clear(1)
        scatter_scan(bufs[1], P32_LO, P5_LO, 1)
        p32 = pltpu.make_async_copy(
            bufs[1].at[pl.ds(0, 32), :],
            out_hbm.at[pl.ds(P32_LO, 32), :], sems[1])
        p32.start()
        scatter_scan(tail5, P5_LO, WIDTH, None)
        p5 = pltpu.make_async_copy(
            tail5, out_hbm.at[pl.ds(P5_LO, WIDTH - P5_LO), :], tsem)
        p5.start()
        fire(0, slab_lo + 16 * CH).wait()
        p32.wait()
        p5.wait()


@jax.jit
def kernel(x_num, cat_idx, mean, std):
    xt_flat = x_num.T.reshape(-1)
    cat_t = jnp.pad(cat_idx.astype(jnp.int32).T, ((0, 6), (0, 0)))
    catt_flat = cat_t.reshape(-1)
    mean_b = jnp.broadcast_to(mean[:, None], (N_NUMERIC, L)).reshape(-1)
    std_b = jnp.broadcast_to(std[:, None], (N_NUMERIC, L)).reshape(-1)

    mesh = plsc.VectorSubcoreMesh(core_axis_name="c", subcore_axis_name="s")
    f = pl.kernel(
        _sc_body,
        out_type=jax.ShapeDtypeStruct((WIDTH, BATCH), jnp.float32),
        mesh=mesh,
        compiler_params=pltpu.CompilerParams(needs_layout_passes=False,
                                             use_tc_tiling_on_sc=True),
        scratch_types=[
            pltpu.VMEM((BATCH,), jnp.float32),
            pltpu.VMEM((2 * BATCH,), jnp.int32),
            pltpu.VMEM((N_NUMERIC * L,), jnp.float32),
            pltpu.VMEM((N_NUMERIC * L,), jnp.float32),
            pltpu.VMEM((NB, 2 * NJ, L), jnp.int32),
            pltpu.VMEM((5, BATCH), jnp.float32),
            [pltpu.VMEM((CH, BATCH), jnp.float32) for _ in range(NB)]
            + [pltpu.SemaphoreType.DMA for _ in range(NB)]
            + [pltpu.SemaphoreType.DMA],
        ],
    )
    return f(xt_flat, catt_flat, mean_b, std_b).T


# speedup vs baseline: 3.2620x; 1.0260x over previous
import jax
import jax.numpy as jnp
from jax import lax
from jax.experimental import pallas as pl
from jax.experimental.pallas import tpu as pltpu
from jax.experimental.pallas import tpu_sc as plsc

NUM_TOKENS = 1000
N_FIELDS = 26
N_NUMERIC = 13
BATCH = 1024
WIDTH = N_NUMERIC + N_FIELDS * NUM_TOKENS
L = 16
NW = 32
SLAB = 816
CH = 40
NFULL = 20
REM = SLAB - NFULL * CH
NJ = BATCH // L
UNROLL = 4
W31_FULL = 17
P32_LO = (NW - 1) * SLAB + W31_FULL * CH
P5_LO = P32_LO + 32
NB = 2


def _sc_body(xt_hbm, catt_hbm, mean_hbm, std_hbm, out_hbm,
             xrow_v, cat_v, m_s, s_s, stash, tail5, bufs_and_sems):
    bufs = bufs_and_sems[:NB]
    sems = bufs_and_sems[NB:NB + NB]
    tsem = bufs_and_sems[2 * NB]
    wid = lax.axis_index("s") * 2 + lax.axis_index("c")
    slab_lo = wid * SLAB

    f0 = jnp.maximum((slab_lo - N_NUMERIC) // NUM_TOKENS, 0)
    pltpu.sync_copy(catt_hbm.at[pl.ds(f0 * BATCH, 2 * BATCH)], cat_v)
    pltpu.sync_copy(mean_hbm, m_s)
    pltpu.sync_copy(std_hbm, s_s)

    zeros = jnp.zeros((L,), jnp.float32)
    ones = jnp.ones((L,), jnp.float32)
    iota = lax.iota(jnp.int32, L)

    def _zero(i, _):
        r = i >> 6
        j = (i & 63) * L
        for b in range(NB):
            bufs[b][r, pl.ds(j, L)] = zeros
        return 0
    lax.fori_loop(0, CH * NJ, _zero, 0)

    def _zero5(i, _):
        tail5[i >> 6, pl.ds((i & 63) * L, L)] = zeros
        return 0
    lax.fori_loop(0, 5 * NJ, _zero5, 0)

    def fld_of(lo):
        return jnp.maximum((lo - N_NUMERIC) // NUM_TOKENS, 0)

    def straddles(lo, hi):
        return fld_of(lo) != fld_of(hi - 1)

    def numeric(b, build):
        @pl.when(wid == 0)
        def _():
            def _row(c, _):
                if build:
                    pltpu.sync_copy(xt_hbm.at[pl.ds(c * BATCH, BATCH)],
                                    xrow_v)
                mc = m_s[pl.ds(c * L, L)]
                sc = s_s[pl.ds(c * L, L)]

                def _col(jq, _):
                    for u in range(UNROLL):
                        j = jq * UNROLL + u
                        if build:
                            v = (xrow_v[pl.ds(j * L, L)] - mc) / sc
                        else:
                            v = zeros
                        bufs[b][c, pl.ds(j * L, L)] = v
                    return 0
                lax.fori_loop(0, NJ // UNROLL, _col, 0)
                return 0
            lax.fori_loop(0, N_NUMERIC, _row, 0)

    def scan(buf, lo, hi, l, stash_plane, prev_plane):
        cbase = N_NUMERIC + (f0 + l) * NUM_TOKENS

        def _s(jq, _):
            for u in range(UNROLL):
                j = jq * UNROLL + u
                if prev_plane is not None:
                    plsc.store_scatter(
                        buf, [prev_plane[j], j * L + iota], zeros)
                c = cbase + cat_v[pl.ds(l * BATCH + j * L, L)]
                m = (c >= lo) & (c < hi)
                krow = jnp.where(m, c - lo, 0)
                if stash_plane is not None:
                    stash_plane[j] = krow
                plsc.store_scatter(buf, [krow, j * L + iota], ones, mask=m)
            return 0
        lax.fori_loop(0, NJ // UNROLL, _s, 0)

    class Plane:
        def __init__(self, b, p):
            self.b, self.p = b, p

        def __getitem__(self, j):
            return stash[self.b, self.p * NJ + j, :]

        def __setitem__(self, j, v):
            stash[self.b, self.p * NJ + j, :] = v

    def clear_plane(buf, plane):
        def _c(jq, _):
            for u in range(UNROLL):
                j = jq * UNROLL + u
                plsc.store_scatter(buf, [plane[j], j * L + iota], zeros)
            return 0
        lax.fori_loop(0, NJ // UNROLL, _c, 0)

    def chunk(b, lo, hi, prev_lo, prev_hi, buf=None):
        tgt = bufs[b] if buf is None else buf
        p0, p1 = Plane(b, 0), Plane(b, 1)
        if prev_lo is not None:
            clear_plane(tgt, p0)
            clear_plane(tgt, p1)
        scan(tgt, lo, hi, 0, p0, None)
        scan(tgt, lo, hi, 1, p1, None)

    def fire(b, lo):
        return pltpu.make_async_copy(
            bufs[b], out_hbm.at[pl.ds(lo, CH), :], sems[b])

    def step(k, b, after_wait=None):
        lo = slab_lo + k * CH
        fire(b, lo - NB * CH).wait()
        if after_wait is not None:
            after_wait()
        chunk(b, lo, lo + CH, lo - NB * CH, lo - NB * CH + CH)
        fire(b, lo).start()

    numeric(0, build=True)
    chunk(0, slab_lo, slab_lo + CH, None, None)
    fire(0, slab_lo).start()
    chunk(1, slab_lo + CH, slab_lo + 2 * CH, None, None)
    fire(1, slab_lo + CH).start()

    def _pair(k2, _):
        def _numclear():
            @pl.when(k2 == 1)
            def _():
                numeric(0, build=False)
        step(2 * k2, 0, after_wait=_numclear)
        step(2 * k2 + 1, 1)
        return 0
    lax.fori_loop(1, 8, _pair, 0)

    @pl.when(wid < NW - 1)
    def _():
        for k in range(16, NFULL):
            step(k, k & 1)
        lo = slab_lo + NFULL * CH
        plo = lo - NB * CH
        fire(0, plo).wait()
        chunk(0, lo, lo + REM, plo, plo + CH)
        rem = pltpu.make_async_copy(
            bufs[0].at[pl.ds(0, REM), :],
            out_hbm.at[pl.ds(lo, REM), :], sems[0])
        rem.start()
        fire(1, slab_lo + (NFULL - 1) * CH).wait()
        rem.wait()

    @pl.when(wid == NW - 1)
    def _():
        step(16, 0)
        plo = slab_lo + 15 * CH
        fire(1, plo).wait()
        chunk(1, P32_LO, P5_LO, plo, plo + CH)
        p32 = pltpu.make_async_copy(
            bufs[1].at[pl.ds(0, 32), :],
            out_hbm.at[pl.ds(P32_LO, 32), :], sems[1])
        p32.start()
        lcur = fld_of(P5_LO) - f0
        scan(tail5, P5_LO, WIDTH, lcur, None, None)
        p5 = pltpu.make_async_copy(
            tail5, out_hbm.at[pl.ds(P5_LO, WIDTH - P5_LO), :], tsem)
        p5.start()
        fire(0, slab_lo + 16 * CH).wait()
        p32.wait()
        p5.wait()


@jax.jit
def kernel(x_num, cat_idx, mean, std):
    xt_flat = x_num.T.reshape(-1)
    cat_t = jnp.pad(cat_idx.astype(jnp.int32).T, ((0, 6), (0, 0)))
    catt_flat = cat_t.reshape(-1)
    mean_b = jnp.broadcast_to(mean[:, None], (N_NUMERIC, L)).reshape(-1)
    std_b = jnp.broadcast_to(std[:, None], (N_NUMERIC, L)).reshape(-1)

    mesh = plsc.VectorSubcoreMesh(core_axis_name="c", subcore_axis_name="s")
    f = pl.kernel(
        _sc_body,
        out_type=jax.ShapeDtypeStruct((WIDTH, BATCH), jnp.float32),
        mesh=mesh,
        compiler_params=pltpu.CompilerParams(needs_layout_passes=False,
                                             use_tc_tiling_on_sc=True),
        scratch_types=[
            pltpu.VMEM((BATCH,), jnp.float32),
            pltpu.VMEM((2 * BATCH,), jnp.int32),
            pltpu.VMEM((N_NUMERIC * L,), jnp.float32),
            pltpu.VMEM((N_NUMERIC * L,), jnp.float32),
            pltpu.VMEM((NB, 2 * NJ, L), jnp.int32),
            pltpu.VMEM((5, BATCH), jnp.float32),
            [pltpu.VMEM((CH, BATCH), jnp.float32) for _ in range(NB)]
            + [pltpu.SemaphoreType.DMA for _ in range(NB)]
            + [pltpu.SemaphoreType.DMA],
        ],
    )
    return f(xt_flat, catt_flat, mean_b, std_b).T


# speedup vs baseline: 3.8142x; 1.1693x over previous
import jax
import jax.numpy as jnp
from jax import lax
from jax.experimental import pallas as pl
from jax.experimental.pallas import tpu as pltpu
from jax.experimental.pallas import tpu_sc as plsc

NUM_TOKENS = 1000
N_FIELDS = 26
N_NUMERIC = 13
BATCH = 1024
WIDTH = N_NUMERIC + N_FIELDS * NUM_TOKENS
L = 16
NW = 32
SLAB = 816
CH = 40
NFULL = 20
REM = SLAB - NFULL * CH
NJ = BATCH // L
UNROLL = 4
W31_FULL = 17
P32_LO = (NW - 1) * SLAB + W31_FULL * CH
P5_LO = P32_LO + 32
NB = 2


def _sc_body(xt_hbm, catt_hbm, mean_hbm, std_hbm, out_hbm,
             xrow_v, cat_v, m_s, s_s, stash, tail5, bufs_and_sems):
    bufs = bufs_and_sems[:NB]
    sems = bufs_and_sems[NB:NB + NB]
    tsem = bufs_and_sems[2 * NB]
    wid = lax.axis_index("s") * 2 + lax.axis_index("c")
    slab_lo = wid * SLAB

    f0 = jnp.maximum((slab_lo - N_NUMERIC) // NUM_TOKENS, 0)
    pltpu.sync_copy(catt_hbm.at[pl.ds(f0 * BATCH, 2 * BATCH)], cat_v)
    pltpu.sync_copy(mean_hbm, m_s)
    pltpu.sync_copy(std_hbm, s_s)

    zeros = jnp.zeros((L,), jnp.float32)
    ones = jnp.ones((L,), jnp.float32)
    iota = lax.iota(jnp.int32, L)

    def _zero(i, _):
        r = i >> 6
        j = (i & 63) * L
        for b in range(NB):
            bufs[b][r, pl.ds(j, L)] = zeros
        return 0
    lax.fori_loop(0, CH * NJ, _zero, 0)

    def _zero5(i, _):
        tail5[i >> 6, pl.ds((i & 63) * L, L)] = zeros
        return 0
    lax.fori_loop(0, 5 * NJ, _zero5, 0)

    def fld_of(lo):
        return jnp.maximum((lo - N_NUMERIC) // NUM_TOKENS, 0)

    def straddles(lo, hi):
        return fld_of(lo) != fld_of(hi - 1)

    def numeric(b, build):
        @pl.when(wid == 0)
        def _():
            def _row(c, _):
                if build:
                    pltpu.sync_copy(xt_hbm.at[pl.ds(c * BATCH, BATCH)],
                                    xrow_v)
                mc = m_s[pl.ds(c * L, L)]
                sc = s_s[pl.ds(c * L, L)]

                def _col(jq, _):
                    for u in range(UNROLL):
                        j = jq * UNROLL + u
                        if build:
                            v = (xrow_v[pl.ds(j * L, L)] - mc) / sc
                        else:
                            v = zeros
                        bufs[b][c, pl.ds(j * L, L)] = v
                    return 0
                lax.fori_loop(0, NJ // UNROLL, _col, 0)
                return 0
            lax.fori_loop(0, N_NUMERIC, _row, 0)

    def scan(buf, lo, hi, l, stash_plane, prev_plane):
        cbase = N_NUMERIC + (f0 + l) * NUM_TOKENS

        def _s(jq, _):
            for u in range(UNROLL):
                j = jq * UNROLL + u
                if prev_plane is not None:
                    plsc.store_scatter(
                        buf, [prev_plane[j], j * L + iota], zeros)
                c = cbase + cat_v[pl.ds(l * BATCH + j * L, L)]
                m = (c >= lo) & (c < hi)
                krow = jnp.where(m, c - lo, 0)
                if stash_plane is not None:
                    stash_plane[j] = krow
                plsc.store_scatter(buf, [krow, j * L + iota], ones, mask=m)
            return 0
        lax.fori_loop(0, NJ // UNROLL, _s, 0)

    class Plane:
        def __init__(self, b, p):
            self.b, self.p = b, p

        def __getitem__(self, j):
            return stash[self.b, self.p * NJ + j, :]

        def __setitem__(self, j, v):
            stash[self.b, self.p * NJ + j, :] = v

    def clear_plane(buf, plane):
        def _c(jq, _):
            for u in range(UNROLL):
                j = jq * UNROLL + u
                plsc.store_scatter(buf, [plane[j], j * L + iota], zeros)
            return 0
        lax.fori_loop(0, NJ // UNROLL, _c, 0)

    def chunk(b, lo, hi, prev_lo, prev_hi, buf=None):
        tgt = bufs[b] if buf is None else buf
        lcur = fld_of(lo) - f0
        p0, p1 = Plane(b, 0), Plane(b, 1)
        if prev_lo is not None:
            clear_plane(tgt, p0)

            @pl.when(straddles(prev_lo, prev_hi))
            def _():
                clear_plane(tgt, p1)

        scan(tgt, lo, hi, lcur, p0, None)

        @pl.when(straddles(lo, hi))
        def _():
            scan(tgt, lo, hi, lcur + 1, p1, None)

    def fire(b, lo):
        return pltpu.make_async_copy(
            bufs[b], out_hbm.at[pl.ds(lo, CH), :], sems[b])

    def step(k, b, after_wait=None):
        lo = slab_lo + k * CH
        fire(b, lo - NB * CH).wait()
        if after_wait is not None:
            after_wait()
        chunk(b, lo, lo + CH, lo - NB * CH, lo - NB * CH + CH)
        fire(b, lo).start()

    numeric(0, build=True)
    chunk(0, slab_lo, slab_lo + CH, None, None)
    fire(0, slab_lo).start()
    chunk(1, slab_lo + CH, slab_lo + 2 * CH, None, None)
    fire(1, slab_lo + CH).start()

    def _pair(k2, _):
        def _numclear():
            @pl.when(k2 == 1)
            def _():
                numeric(0, build=False)
        step(2 * k2, 0, after_wait=_numclear)
        step(2 * k2 + 1, 1)
        return 0
    lax.fori_loop(1, 8, _pair, 0)

    @pl.when(wid < NW - 1)
    def _():
        for k in range(16, NFULL):
            step(k, k & 1)
        lo = slab_lo + NFULL * CH
        plo = lo - NB * CH
        fire(0, plo).wait()
        chunk(0, lo, lo + REM, plo, plo + CH)
        rem = pltpu.make_async_copy(
            bufs[0].at[pl.ds(0, REM), :],
            out_hbm.at[pl.ds(lo, REM), :], sems[0])
        rem.start()
        fire(1, slab_lo + (NFULL - 1) * CH).wait()
        rem.wait()

    @pl.when(wid == NW - 1)
    def _():
        step(16, 0)
        plo = slab_lo + 15 * CH
        fire(1, plo).wait()
        chunk(1, P32_LO, P5_LO, plo, plo + CH)
        p32 = pltpu.make_async_copy(
            bufs[1].at[pl.ds(0, 32), :],
            out_hbm.at[pl.ds(P32_LO, 32), :], sems[1])
        p32.start()
        lcur = fld_of(P5_LO) - f0
        scan(tail5, P5_LO, WIDTH, lcur, None, None)
        p5 = pltpu.make_async_copy(
            tail5, out_hbm.at[pl.ds(P5_LO, WIDTH - P5_LO), :], tsem)
        p5.start()
        fire(0, slab_lo + 16 * CH).wait()
        p32.wait()
        p5.wait()


@jax.jit
def kernel(x_num, cat_idx, mean, std):
    xt_flat = x_num.T.reshape(-1)
    cat_t = jnp.pad(cat_idx.astype(jnp.int32).T, ((0, 6), (0, 0)))
    catt_flat = cat_t.reshape(-1)
    mean_b = jnp.broadcast_to(mean[:, None], (N_NUMERIC, L)).reshape(-1)
    std_b = jnp.broadcast_to(std[:, None], (N_NUMERIC, L)).reshape(-1)

    mesh = plsc.VectorSubcoreMesh(core_axis_name="c", subcore_axis_name="s")
    f = pl.kernel(
        _sc_body,
        out_type=jax.ShapeDtypeStruct((WIDTH, BATCH), jnp.float32),
        mesh=mesh,
        compiler_params=pltpu.CompilerParams(needs_layout_passes=False,
                                             use_tc_tiling_on_sc=True),
        scratch_types=[
            pltpu.VMEM((BATCH,), jnp.float32),
            pltpu.VMEM((2 * BATCH,), jnp.int32),
            pltpu.VMEM((N_NUMERIC * L,), jnp.float32),
            pltpu.VMEM((N_NUMERIC * L,), jnp.float32),
            pltpu.VMEM((NB, 2 * NJ, L), jnp.int32),
            pltpu.VMEM((5, BATCH), jnp.float32),
            [pltpu.VMEM((CH, BATCH), jnp.float32) for _ in range(NB)]
            + [pltpu.SemaphoreType.DMA for _ in range(NB)]
            + [pltpu.SemaphoreType.DMA],
        ],
    )
    return f(xt_flat, catt_flat, mean_b, std_b).T


# speedup vs baseline: 4.6007x; 1.2062x over previous
import jax
import jax.numpy as jnp
from jax import lax
from jax.experimental import pallas as pl
from jax.experimental.pallas import tpu as pltpu
from jax.experimental.pallas import tpu_sc as plsc

NUM_TOKENS = 1000
N_FIELDS = 26
N_NUMERIC = 13
BATCH = 1024
WIDTH = N_NUMERIC + N_FIELDS * NUM_TOKENS
L = 16
NW = 32
SLAB = 816
CH = 40
NFULL = 20
REM = SLAB - NFULL * CH
NJ = BATCH // L
UNROLL = 4
W31_FULL = 17
P32_LO = (NW - 1) * SLAB + W31_FULL * CH
P5_LO = P32_LO + 32
NB = 2


def _sc_body(xt_hbm, catt_hbm, mean_hbm, std_hbm, out_hbm,
             cat_v, m_s, s_s, stash, tail5, bufs_and_sems):
    bufs = bufs_and_sems[:NB]
    sems = bufs_and_sems[NB:NB + NB]
    tsem = bufs_and_sems[2 * NB]
    wid = lax.axis_index("s") * 2 + lax.axis_index("c")
    slab_lo = wid * SLAB

    f0 = jnp.maximum((slab_lo - N_NUMERIC) // NUM_TOKENS, 0)
    pltpu.sync_copy(catt_hbm.at[pl.ds(f0 * BATCH, 2 * BATCH)], cat_v)
    pltpu.sync_copy(mean_hbm, m_s)
    pltpu.sync_copy(std_hbm, s_s)

    zeros = jnp.zeros((L,), jnp.float32)
    ones = jnp.ones((L,), jnp.float32)
    iota = lax.iota(jnp.int32, L)

    def _zero(i, _):
        r = i >> 6
        j = (i & 63) * L
        for b in range(NB):
            bufs[b][r, pl.ds(j, L)] = zeros
        return 0
    lax.fori_loop(0, CH * NJ, _zero, 0)

    def _zero5(i, _):
        tail5[i >> 6, pl.ds((i & 63) * L, L)] = zeros
        return 0
    lax.fori_loop(0, 5 * NJ, _zero5, 0)

    def fld_of(lo):
        return jnp.maximum((lo - N_NUMERIC) // NUM_TOKENS, 0)

    def straddles(lo, hi):
        return fld_of(lo) != fld_of(hi - 1)

    def numeric(b, build):
        @pl.when(wid == 0)
        def _():
            if build:
                pltpu.sync_copy(xt_hbm, bufs[1].at[pl.ds(0, 16), :])

            def _row(c, _):
                mc = m_s[pl.ds(c * L, L)]
                rc = ones / s_s[pl.ds(c * L, L)]

                def _col(jq, _):
                    for u in range(UNROLL):
                        j = jq * UNROLL + u
                        if build:
                            v = (bufs[1][c, pl.ds(j * L, L)] - mc) * rc
                        else:
                            v = zeros
                        bufs[b][c, pl.ds(j * L, L)] = v
                    return 0
                lax.fori_loop(0, NJ // UNROLL, _col, 0)
                return 0
            lax.fori_loop(0, N_NUMERIC, _row, 0)
            if build:
                def _rz(i, _):
                    for u in range(UNROLL):
                        bufs[1][i >> 4, pl.ds(((i & 15) * 4 + u) * L, L)] = \
                            zeros
                    return 0
                lax.fori_loop(0, 16 * 16, _rz, 0)

    def scan(buf, lo, hi, l, stash_plane, prev_plane):
        cbase = N_NUMERIC + (f0 + l) * NUM_TOKENS

        def _s(jq, _):
            for u in range(UNROLL):
                j = jq * UNROLL + u
                if prev_plane is not None:
                    plsc.store_scatter(
                        buf, [prev_plane[j], j * L + iota], zeros)
                c = cbase + cat_v[pl.ds(l * BATCH + j * L, L)]
                m = (c >= lo) & (c < hi)
                krow = jnp.where(m, c - lo, 0)
                if stash_plane is not None:
                    stash_plane[j] = krow
                plsc.store_scatter(buf, [krow, j * L + iota], ones, mask=m)
            return 0
        lax.fori_loop(0, NJ // UNROLL, _s, 0)

    class Plane:
        def __init__(self, b, p):
            self.b, self.p = b, p

        def __getitem__(self, j):
            return stash[self.b, self.p * NJ + j, :]

        def __setitem__(self, j, v):
            stash[self.b, self.p * NJ + j, :] = v

    def clear_plane(buf, plane):
        def _c(jq, _):
            for u in range(UNROLL):
                j = jq * UNROLL + u
                plsc.store_scatter(buf, [plane[j], j * L + iota], zeros)
            return 0
        lax.fori_loop(0, NJ // UNROLL, _c, 0)

    def chunk(b, lo, hi, prev_lo, prev_hi, buf=None):
        tgt = bufs[b] if buf is None else buf
        lcur = fld_of(lo) - f0
        p0, p1 = Plane(b, 0), Plane(b, 1)
        if prev_lo is not None:
            clear_plane(tgt, p0)

            @pl.when(straddles(prev_lo, prev_hi))
            def _():
                clear_plane(tgt, p1)

        scan(tgt, lo, hi, lcur, p0, None)

        @pl.when(straddles(lo, hi))
        def _():
            scan(tgt, lo, hi, lcur + 1, p1, None)

    def fire(b, lo):
        return pltpu.make_async_copy(
            bufs[b], out_hbm.at[pl.ds(lo, CH), :], sems[b])

    def step(k, b, after_wait=None):
        lo = slab_lo + k * CH
        fire(b, lo - NB * CH).wait()
        if after_wait is not None:
            after_wait()
        chunk(b, lo, lo + CH, lo - NB * CH, lo - NB * CH + CH)
        fire(b, lo).start()

    numeric(0, build=True)
    chunk(0, slab_lo, slab_lo + CH, None, None)
    fire(0, slab_lo).start()
    chunk(1, slab_lo + CH, slab_lo + 2 * CH, None, None)
    fire(1, slab_lo + CH).start()

    def _pair(k2, _):
        def _numclear():
            @pl.when(k2 == 1)
            def _():
                numeric(0, build=False)
        step(2 * k2, 0, after_wait=_numclear)
        step(2 * k2 + 1, 1)
        return 0
    lax.fori_loop(1, 8, _pair, 0)

    @pl.when(wid < NW - 1)
    def _():
        for k in range(16, NFULL):
            step(k, k & 1)
        lo = slab_lo + NFULL * CH
        plo = lo - NB * CH
        fire(0, plo).wait()
        chunk(0, lo, lo + REM, plo, plo + CH)
        rem = pltpu.make_async_copy(
            bufs[0].at[pl.ds(0, REM), :],
            out_hbm.at[pl.ds(lo, REM), :], sems[0])
        rem.start()
        fire(1, slab_lo + (NFULL - 1) * CH).wait()
        rem.wait()

    @pl.when(wid == NW - 1)
    def _():
        step(16, 0)
        plo = slab_lo + 15 * CH
        fire(1, plo).wait()
        chunk(1, P32_LO, P5_LO, plo, plo + CH)
        p32 = pltpu.make_async_copy(
            bufs[1].at[pl.ds(0, 32), :],
            out_hbm.at[pl.ds(P32_LO, 32), :], sems[1])
        p32.start()
        lcur = fld_of(P5_LO) - f0
        scan(tail5, P5_LO, WIDTH, lcur, None, None)
        p5 = pltpu.make_async_copy(
            tail5, out_hbm.at[pl.ds(P5_LO, WIDTH - P5_LO), :], tsem)
        p5.start()
        fire(0, slab_lo + 16 * CH).wait()
        p32.wait()
        p5.wait()


@jax.jit
def kernel(x_num, cat_idx, mean, std):
    xt_pad = jnp.pad(x_num.T, ((0, 3), (0, 0)))
    cat_t = jnp.pad(cat_idx.astype(jnp.int32).T, ((0, 6), (0, 0)))
    catt_flat = cat_t.reshape(-1)
    mean_b = jnp.broadcast_to(mean[:, None], (N_NUMERIC, L)).reshape(-1)
    std_b = jnp.broadcast_to(std[:, None], (N_NUMERIC, L)).reshape(-1)

    mesh = plsc.VectorSubcoreMesh(core_axis_name="c", subcore_axis_name="s")
    f = pl.kernel(
        _sc_body,
        out_type=jax.ShapeDtypeStruct((WIDTH, BATCH), jnp.float32),
        mesh=mesh,
        compiler_params=pltpu.CompilerParams(needs_layout_passes=False,
                                             use_tc_tiling_on_sc=True),
        scratch_types=[
            pltpu.VMEM((2 * BATCH,), jnp.int32),
            pltpu.VMEM((N_NUMERIC * L,), jnp.float32),
            pltpu.VMEM((N_NUMERIC * L,), jnp.float32),
            pltpu.VMEM((NB, 2 * NJ, L), jnp.int32),
            pltpu.VMEM((5, BATCH), jnp.float32),
            [pltpu.VMEM((CH, BATCH), jnp.float32) for _ in range(NB)]
            + [pltpu.SemaphoreType.DMA for _ in range(NB)]
            + [pltpu.SemaphoreType.DMA],
        ],
    )
    return f(xt_pad, catt_flat, mean_b, std_b).T


# speedup vs baseline: 5.1910x; 1.1283x over previous
import jax
import jax.numpy as jnp
from jax import lax
from jax.experimental import pallas as pl
from jax.experimental.pallas import tpu as pltpu
from jax.experimental.pallas import tpu_sc as plsc

NUM_TOKENS = 1000
N_FIELDS = 26
N_NUMERIC = 13
BATCH = 1024
WIDTH = N_NUMERIC + N_FIELDS * NUM_TOKENS
L = 16
NW = 32
SLAB = 816
CH = 40
NFULL = 20
REM = SLAB - NFULL * CH
NJ = BATCH // L
UNROLL = 4
W31_FULL = 17
P32_LO = (NW - 1) * SLAB + W31_FULL * CH
P5_LO = P32_LO + 32
NB = 2


def _sc_body(xt_hbm, catt_hbm, mean_hbm, std_hbm, out_hbm,
             cat_v, m_s, s_s, stash, tail5, bufs_and_sems):
    bufs = bufs_and_sems[:NB]
    sems = bufs_and_sems[NB:NB + NB]
    tsem = bufs_and_sems[2 * NB]
    wid = lax.axis_index("s") * 2 + lax.axis_index("c")
    slab_lo = wid * SLAB

    f0 = jnp.maximum((slab_lo - N_NUMERIC) // NUM_TOKENS, 0)
    pltpu.sync_copy(catt_hbm.at[pl.ds(f0 * BATCH, 2 * BATCH)], cat_v)
    pltpu.sync_copy(mean_hbm, m_s)
    pltpu.sync_copy(std_hbm, s_s)

    zeros = jnp.zeros((L,), jnp.float32)
    ones = jnp.ones((L,), jnp.float32)
    iota = lax.iota(jnp.int32, L)

    def _zero(i, _):
        r = i >> 3
        for b in range(NB):
            for u in range(8):
                bufs[b][r, pl.ds(((i & 7) * 8 + u) * L, L)] = zeros
        return 0
    lax.fori_loop(0, CH * 8, _zero, 0)

    def _zero5(i, _):
        for u in range(8):
            tail5[i >> 3, pl.ds(((i & 7) * 8 + u) * L, L)] = zeros
        return 0
    lax.fori_loop(0, 5 * 8, _zero5, 0)

    def fld_of(lo):
        return jnp.maximum((lo - N_NUMERIC) // NUM_TOKENS, 0)

    def straddles(lo, hi):
        return fld_of(lo) != fld_of(hi - 1)

    def numeric(b, build):
        @pl.when(wid == 0)
        def _():
            if build:
                pltpu.sync_copy(xt_hbm, bufs[1].at[pl.ds(0, 16), :])

            def _row(c, _):
                mc = m_s[pl.ds(c * L, L)]
                rc = ones / s_s[pl.ds(c * L, L)]

                def _col(jq, _):
                    for u in range(UNROLL):
                        j = jq * UNROLL + u
                        if build:
                            v = (bufs[1][c, pl.ds(j * L, L)] - mc) * rc
                        else:
                            v = zeros
                        bufs[b][c, pl.ds(j * L, L)] = v
                    return 0
                lax.fori_loop(0, NJ // UNROLL, _col, 0)
                return 0
            lax.fori_loop(0, N_NUMERIC, _row, 0)
            if build:
                def _rz(i, _):
                    for u in range(UNROLL):
                        bufs[1][i >> 4, pl.ds(((i & 15) * 4 + u) * L, L)] = \
                            zeros
                    return 0
                lax.fori_loop(0, 16 * 16, _rz, 0)

    def scan(buf, lo, hi, l, stash_plane, prev_plane):
        cbase = N_NUMERIC + (f0 + l) * NUM_TOKENS

        def _s(jq, _):
            for u in range(UNROLL):
                j = jq * UNROLL + u
                if prev_plane is not None:
                    plsc.store_scatter(
                        buf, [prev_plane[j], j * L + iota], zeros)
                c = cbase + cat_v[pl.ds(l * BATCH + j * L, L)]
                m = (c >= lo) & (c < hi)
                krow = jnp.where(m, c - lo, 0)
                if stash_plane is not None:
                    stash_plane[j] = krow
                plsc.store_scatter(buf, [krow, j * L + iota], ones, mask=m)
            return 0
        lax.fori_loop(0, NJ // UNROLL, _s, 0)

    class Plane:
        def __init__(self, b, p):
            self.b, self.p = b, p

        def __getitem__(self, j):
            return stash[self.b, self.p * NJ + j, :]

        def __setitem__(self, j, v):
            stash[self.b, self.p * NJ + j, :] = v

    def clear_plane(buf, plane):
        def _c(jq, _):
            for u in range(UNROLL):
                j = jq * UNROLL + u
                plsc.store_scatter(buf, [plane[j], j * L + iota], zeros)
            return 0
        lax.fori_loop(0, NJ // UNROLL, _c, 0)

    def chunk(b, lo, hi, prev_lo, prev_hi, buf=None):
        tgt = bufs[b] if buf is None else buf
        lcur = fld_of(lo) - f0
        p0, p1 = Plane(b, 0), Plane(b, 1)
        if prev_lo is not None:
            clear_plane(tgt, p0)

            @pl.when(straddles(prev_lo, prev_hi))
            def _():
                clear_plane(tgt, p1)

        scan(tgt, lo, hi, lcur, p0, None)

        @pl.when(straddles(lo, hi))
        def _():
            scan(tgt, lo, hi, lcur + 1, p1, None)

    def fire(b, lo):
        return pltpu.make_async_copy(
            bufs[b], out_hbm.at[pl.ds(lo, CH), :], sems[b])

    def step(k, b, after_wait=None):
        lo = slab_lo + k * CH
        fire(b, lo - NB * CH).wait()
        if after_wait is not None:
            after_wait()
        chunk(b, lo, lo + CH, lo - NB * CH, lo - NB * CH + CH)
        fire(b, lo).start()

    numeric(0, build=True)
    chunk(0, slab_lo, slab_lo + CH, None, None)
    fire(0, slab_lo).start()
    chunk(1, slab_lo + CH, slab_lo + 2 * CH, None, None)
    fire(1, slab_lo + CH).start()

    def _pair(k2, _):
        def _numclear():
            @pl.when(k2 == 1)
            def _():
                numeric(0, build=False)
        step(2 * k2, 0, after_wait=_numclear)
        step(2 * k2 + 1, 1)
        return 0
    lax.fori_loop(1, 8, _pair, 0)

    @pl.when(wid < NW - 1)
    def _():
        for k in range(16, NFULL):
            step(k, k & 1)
        lo = slab_lo + NFULL * CH
        plo = lo - NB * CH
        fire(0, plo).wait()
        chunk(0, lo, lo + REM, plo, plo + CH)
        rem = pltpu.make_async_copy(
            bufs[0].at[pl.ds(0, REM), :],
            out_hbm.at[pl.ds(lo, REM), :], sems[0])
        rem.start()
        fire(1, slab_lo + (NFULL - 1) * CH).wait()
        rem.wait()

    @pl.when(wid == NW - 1)
    def _():
        step(16, 0)
        plo = slab_lo + 15 * CH
        fire(1, plo).wait()
        chunk(1, P32_LO, P5_LO, plo, plo + CH)
        p32 = pltpu.make_async_copy(
            bufs[1].at[pl.ds(0, 32), :],
            out_hbm.at[pl.ds(P32_LO, 32), :], sems[1])
        p32.start()
        lcur = fld_of(P5_LO) - f0
        scan(tail5, P5_LO, WIDTH, lcur, None, None)
        p5 = pltpu.make_async_copy(
            tail5, out_hbm.at[pl.ds(P5_LO, WIDTH - P5_LO), :], tsem)
        p5.start()
        fire(0, slab_lo + 16 * CH).wait()
        p32.wait()
        p5.wait()


@jax.jit
def kernel(x_num, cat_idx, mean, std):
    xt_pad = jnp.pad(x_num.T, ((0, 3), (0, 0)))
    cat_t = jnp.pad(cat_idx.astype(jnp.int32).T, ((0, 6), (0, 0)))
    catt_flat = cat_t.reshape(-1)
    mean_b = jnp.broadcast_to(mean[:, None], (N_NUMERIC, L)).reshape(-1)
    std_b = jnp.broadcast_to(std[:, None], (N_NUMERIC, L)).reshape(-1)

    mesh = plsc.VectorSubcoreMesh(core_axis_name="c", subcore_axis_name="s")
    f = pl.kernel(
        _sc_body,
        out_type=jax.ShapeDtypeStruct((WIDTH, BATCH), jnp.float32),
        mesh=mesh,
        compiler_params=pltpu.CompilerParams(needs_layout_passes=False,
                                             use_tc_tiling_on_sc=True),
        scratch_types=[
            pltpu.VMEM((2 * BATCH,), jnp.int32),
            pltpu.VMEM((N_NUMERIC * L,), jnp.float32),
            pltpu.VMEM((N_NUMERIC * L,), jnp.float32),
            pltpu.VMEM((NB, 2 * NJ, L), jnp.int32),
            pltpu.VMEM((5, BATCH), jnp.float32),
            [pltpu.VMEM((CH, BATCH), jnp.float32) for _ in range(NB)]
            + [pltpu.SemaphoreType.DMA for _ in range(NB)]
            + [pltpu.SemaphoreType.DMA],
        ],
    )
    return f(xt_pad, catt_flat, mean_b, std_b).T
